# baseline (speedup 1.0000x reference)
import functools

import jax
import jax.numpy as jnp
from jax import lax
from jax.experimental import pallas as pl
from jax.experimental.pallas import tpu as pltpu
from jax.experimental.pallas import tpu_sc as plsc

N = 10000
E = 320000
H = 128
VOCAB = 128
NET = 4
PED = 37
G = 128
L = 4

Np = 10240
NB = 512
NBLK = Np // NB

NC = 2
NS = 16
NWORK = NC * NS
C = 64
CHUNKS = 160
E_PAD = NWORK * C * CHUNKS
NCHT = E_PAD // C
NBUF = 3
GROUPS = CHUNKS // NBUF
TAIL = CHUNKS - GROUPS * NBUF
ROWS_PT = Np // NS


def _sc_edge_body(ipack_hbm, haug_hbm, agg_hbm,
                  idx_all, idx_v, buf_v, agg_s, *sems):
    sem_g = sems[:NBUF]
    sem_s = sems[NBUF:]
    cid = lax.axis_index("c")
    sid = lax.axis_index("s")

    tci = (cid * NS + sid) * CHUNKS
    pltpu.sync_copy(ipack_hbm.at[pl.ds(tci, CHUNKS)], idx_all)

    def _zrow(r, _):
        for j in range(H // 16):
            buf_v[0, r, pl.ds(j * 16, 16)] = jnp.zeros((16,), jnp.float32)
        return 0
    lax.fori_loop(0, C, _zrow, 0)

    def _zcopy(k, _):
        pltpu.sync_copy(buf_v.at[0], agg_s.at[pl.ds(sid * ROWS_PT + k * C, C)])
        return 0
    lax.fori_loop(0, ROWS_PT // C, _zcopy, 0)
    plsc.subcore_barrier()

    def _unpack(k, b):
        for j in range(C // 16):
            w = idx_all[b, pl.ds(j * 16, 16)]
            idx_v[k, 0, pl.ds(j * 16, 16)] = jnp.bitwise_and(w, 0xFFFF)
            idx_v[k, 1, pl.ds(j * 16, 16)] = jnp.right_shift(w, 16)

    def _start_gather(k):
        pltpu.async_copy(haug_hbm.at[idx_v.at[k, 0]], buf_v.at[k], sem_g[k])

    def _drain_scatter(k):
        pltpu.make_async_copy(buf_v.at[k], agg_s.at[idx_v.at[k, 1]],
                              sem_s[k]).wait()

    for k in range(2):
        _unpack(k, k)
        _start_gather(k)

    def _step(k, b, refill):
        pltpu.make_async_copy(haug_hbm.at[idx_v.at[k, 0]], buf_v.at[k],
                              sem_g[k]).wait()

        def _relu(i, _):
            for j in range(H // 16):
                sl = buf_v[k, i, pl.ds(j * 16, 16)]
                buf_v[k, i, pl.ds(j * 16, 16)] = jnp.maximum(sl, 0.0)
            return 0
        lax.fori_loop(0, C, _relu, 0)
        pltpu.async_copy(buf_v.at[k], agg_s.at[idx_v.at[k, 1]],
                         sem_s[k], add=True)

        if refill:
            k2 = (k + 2) % NBUF

            @pl.when(b + 2 < CHUNKS)
            def _():
                @pl.when(b >= NBUF - 2)
                def _():
                    _drain_scatter(k2)
                _unpack(k2, b + 2)
                _start_gather(k2)

    def _group(g, _):
        for k in range(NBUF):
            _step(k, g * NBUF + k, True)
        return 0
    lax.fori_loop(0, GROUPS, _group, 0)
    for t in range(TAIL):
        _step((GROUPS * NBUF + t) % NBUF, GROUPS * NBUF + t, False)
    for t in range(NBUF):
        _drain_scatter((CHUNKS - NBUF + t) % NBUF)

    plsc.subcore_barrier()
    r0 = sid * ROWS_PT
    pltpu.sync_copy(agg_s.at[pl.ds(r0, ROWS_PT)],
                    agg_hbm.at[cid, pl.ds(r0, ROWS_PT)])


@functools.cache
def _get_sc_edge():
    return pl.kernel(
        _sc_edge_body,
        out_type=jax.ShapeDtypeStruct((NC, Np, H), jnp.float32),
        mesh=plsc.VectorSubcoreMesh(core_axis_name="c",
                                    subcore_axis_name="s",
                                    num_cores=NC, num_subcores=NS),
        scratch_types=(
            [pltpu.VMEM((CHUNKS, C), jnp.int32),
             pltpu.VMEM((NBUF, 2, C), jnp.int32),
             pltpu.VMEM((NBUF, C, H), jnp.float32),
             pltpu.VMEM_SHARED((Np, H), jnp.float32)]
            + [pltpu.SemaphoreType.DMA] * (2 * NBUF)
        ),
    )


def _embed_body(xn_ref, pe_ref, emb_ref, pew_ref, peb_ref, ea_ref,
                h_ref, haug_ref):
    x = xn_ref[...]
    iota = lax.broadcasted_iota(jnp.int32, (NB, VOCAB), 1)
    onehot = (x == iota).astype(jnp.float32)
    h = jnp.dot(onehot, emb_ref[...], preferred_element_type=jnp.float32,
                precision=lax.Precision.HIGHEST)
    h += jnp.dot(pe_ref[...], pew_ref[...], preferred_element_type=jnp.float32,
                precision=lax.Precision.HIGHEST)
    h += peb_ref[...]
    h_ref[...] = h
    haug_ref[...] = h[None, :, :] + ea_ref[...][:, None, :]


def _mlp_body_aug(h_ref, agg_ref, w1_ref, b1_ref, w2_ref, b2_ref,
                  sc_ref, ea_ref, ho_ref, haug_ref):
    z = sc_ref[0, 0] * h_ref[...] + agg_ref[0] + agg_ref[1]
    a = jnp.maximum(
        jnp.dot(z, w1_ref[...], preferred_element_type=jnp.float32,
                precision=lax.Precision.HIGHEST)
        + b1_ref[...], 0.0)
    hn = jnp.dot(a, w2_ref[...], preferred_element_type=jnp.float32,
                precision=lax.Precision.HIGHEST) \
        + b2_ref[...]
    ho_ref[...] = hn
    haug_ref[...] = hn[None, :, :] + ea_ref[...][:, None, :]


def _mlp_body_last(h_ref, agg_ref, w1_ref, b1_ref, w2_ref, b2_ref,
                   sc_ref, ho_ref):
    z = sc_ref[0, 0] * h_ref[...] + agg_ref[0] + agg_ref[1]
    a = jnp.maximum(
        jnp.dot(z, w1_ref[...], preferred_element_type=jnp.float32,
                precision=lax.Precision.HIGHEST)
        + b1_ref[...], 0.0)
    ho_ref[...] = jnp.dot(a, w2_ref[...],
                          preferred_element_type=jnp.float32,
                precision=lax.Precision.HIGHEST) + b2_ref[...]


def _pool_head_body(bt_ref, h_ref, f1_ref, fb1_ref, f2_ref, fb2_ref,
                    f3_ref, fb3_ref, y_ref, acc, cnt):
    i = pl.program_id(0)

    @pl.when(i == 0)
    def _init():
        acc[...] = jnp.zeros((G, H), jnp.float32)
        cnt[...] = jnp.zeros((G, 1), jnp.float32)

    b = bt_ref[...]
    iota = lax.broadcasted_iota(jnp.int32, (NB, G), 1)
    m = (b == iota).astype(jnp.float32)
    dn = (((0,), (0,)), ((), ()))
    acc[...] += lax.dot_general(m, h_ref[...], dn,
                                preferred_element_type=jnp.float32,
                precision=lax.Precision.HIGHEST)
    cnt[...] += lax.dot_general(m, jnp.ones((NB, 1), jnp.float32), dn,
                                preferred_element_type=jnp.float32,
                precision=lax.Precision.HIGHEST)

    @pl.when(i == NBLK - 1)
    def _final():
        hp = acc[...] / jnp.maximum(cnt[...], 1.0)
        y = jnp.maximum(
            jnp.dot(hp, f1_ref[...], preferred_element_type=jnp.float32,
                precision=lax.Precision.HIGHEST)
            + fb1_ref[...], 0.0)
        y = jnp.maximum(
            jnp.dot(y, f2_ref[...], preferred_element_type=jnp.float32,
                precision=lax.Precision.HIGHEST)
            + fb2_ref[...], 0.0)
        y_ref[...] = jnp.dot(y, f3_ref[...],
                             preferred_element_type=jnp.float32,
                precision=lax.Precision.HIGHEST) + fb3_ref[...]


def _full(shape):
    return pl.BlockSpec(shape, lambda i: (0,) * len(shape))


_row_i = pl.BlockSpec((NB, 1), lambda i: (i, 0))
_row_h = pl.BlockSpec((NB, H), lambda i: (i, 0))
_aug_b = pl.BlockSpec((NET, NB, H), lambda i: (0, i, 0))
_agg_b = pl.BlockSpec((NC, NB, H), lambda i: (0, i, 0))
_smem1 = pl.BlockSpec(memory_space=pltpu.SMEM)

_embed_call = pl.pallas_call(
    _embed_body,
    grid=(NBLK,),
    in_specs=[_row_i, _row_h, _full((VOCAB, H)), _full((H, H)),
              _full((1, H)), _full((NET, H))],
    out_specs=[_row_h, _aug_b],
    out_shape=[jax.ShapeDtypeStruct((Np, H), jnp.float32),
               jax.ShapeDtypeStruct((NET, Np, H), jnp.float32)],
)

_mlp_aug_call = pl.pallas_call(
    _mlp_body_aug,
    grid=(NBLK,),
    in_specs=[_row_h, _agg_b, _full((H, H)), _full((1, H)), _full((H, H)),
              _full((1, H)), _smem1, _full((NET, H))],
    out_specs=[_row_h, _aug_b],
    out_shape=[jax.ShapeDtypeStruct((Np, H), jnp.float32),
               jax.ShapeDtypeStruct((NET, Np, H), jnp.float32)],
)

_mlp_last_call = pl.pallas_call(
    _mlp_body_last,
    grid=(NBLK,),
    in_specs=[_row_h, _agg_b, _full((H, H)), _full((1, H)), _full((H, H)),
              _full((1, H)), _smem1],
    out_specs=_row_h,
    out_shape=jax.ShapeDtypeStruct((Np, H), jnp.float32),
)

_pool_head_call = pl.pallas_call(
    _pool_head_body,
    grid=(NBLK,),
    in_specs=[_row_i, _row_h, _full((H, H)), _full((1, H)), _full((H, H)),
              _full((1, H)), _full((H, 1)), _full((1, 1))],
    out_specs=pl.BlockSpec((G, 1), lambda i: (0, 0)),
    out_shape=jax.ShapeDtypeStruct((G, 1), jnp.float32),
    scratch_shapes=[pltpu.VMEM((G, H), jnp.float32),
                    pltpu.VMEM((G, 1), jnp.float32)],
)


def kernel(X_n, edge_index, edge_attr, PE, snorm, batch, node_emb, pe_W,
           pe_b, edge_embs, W1, b1, W2, b2, eps, fW1, fb1, fW2, fb2, fW3,
           fb3):
    i32 = jnp.int32
    src = edge_index[0].astype(i32)
    dst = edge_index[1].astype(i32)
    attr = edge_attr.astype(i32)
    gidx = attr * Np + src
    pad_e = E_PAD - E
    gidx = jnp.concatenate([gidx, jnp.zeros((pad_e,), i32)])
    dst = jnp.concatenate([dst, jnp.full((pad_e,), N, i32)])
    ipack = ((dst << 16) | gidx).reshape(NCHT, C)

    xn = jnp.concatenate([X_n.astype(i32),
                          jnp.zeros((Np - N,), i32)]).reshape(Np, 1)
    pe = jnp.zeros((Np, H), jnp.float32).at[:N, :PED].set(PE)
    pew = jnp.zeros((H, H), jnp.float32).at[:PED].set(pe_W)
    bt = jnp.concatenate([batch.astype(i32),
                          jnp.full((Np - N,), G, i32)]).reshape(Np, 1)

    h, haug = _embed_call(xn, pe, node_emb, pew, pe_b.reshape(1, H),
                          edge_embs[0])
    for l in range(L):
        agg = _get_sc_edge()(ipack, haug.reshape(NET * Np, H))
        scale = (1.0 + eps[l]).reshape(1, 1)
        if l + 1 < L:
            h, haug = _mlp_aug_call(h, agg, W1[l], b1[l].reshape(1, H),
                                    W2[l], b2[l].reshape(1, H), scale,
                                    edge_embs[l + 1])
        else:
            h = _mlp_last_call(h, agg, W1[l], b1[l].reshape(1, H), W2[l],
                               b2[l].reshape(1, H), scale)

    y = _pool_head_call(bt, h, fW1, fb1.reshape(1, H), fW2,
                        fb2.reshape(1, H), fW3, fb3.reshape(1, 1))
    return y[:, 0]


# speedup vs baseline: 1.0856x; 1.0856x over previous
import functools

import jax
import jax.numpy as jnp
from jax import lax
from jax.experimental import pallas as pl
from jax.experimental.pallas import tpu as pltpu
from jax.experimental.pallas import tpu_sc as plsc

N = 10000
E = 320000
H = 128
VOCAB = 128
NET = 4
PED = 37
G = 128
L = 4

Np = 10240
NB = 512
NBLK = Np // NB

NC = 2
NS = 16
NWORK = NC * NS
C = 64
CHUNKS = 160
E_PAD = NWORK * C * CHUNKS
NCHT = E_PAD // C
NBUF = 3
GROUPS = CHUNKS // NBUF
TAIL = CHUNKS - GROUPS * NBUF
ROWS_PT = Np // NS


def _sc_edge_body(ipack_hbm, haug_hbm, agg_hbm,
                  idx_all, idx_v, buf_v, agg_s, *sems):
    sem_g = sems[:NBUF]
    sem_g2 = sems[NBUF:2 * NBUF]
    sem_s = sems[2 * NBUF:]
    cid = lax.axis_index("c")
    sid = lax.axis_index("s")

    tci = (cid * NS + sid) * CHUNKS
    pltpu.sync_copy(ipack_hbm.at[pl.ds(tci, CHUNKS)], idx_all)

    def _zrow(r, _):
        for j in range(H // 16):
            buf_v[0, r, pl.ds(j * 16, 16)] = jnp.zeros((16,), jnp.float32)
        return 0
    lax.fori_loop(0, C, _zrow, 0)

    def _zcopy(k, _):
        pltpu.sync_copy(buf_v.at[0], agg_s.at[pl.ds(sid * ROWS_PT + k * C, C)])
        return 0
    lax.fori_loop(0, ROWS_PT // C, _zcopy, 0)
    plsc.subcore_barrier()

    def _unpack(k, b):
        for j in range(C // 16):
            w = idx_all[b, pl.ds(j * 16, 16)]
            idx_v[k, 0, pl.ds(j * 16, 16)] = jnp.bitwise_and(w, 0xFFFF)
            idx_v[k, 1, pl.ds(j * 16, 16)] = jnp.right_shift(w, 16)

    def _start_gather(k):
        hc = C // 2
        pltpu.async_copy(haug_hbm.at[idx_v.at[k, 0, pl.ds(0, hc)]],
                         buf_v.at[k, pl.ds(0, hc)], sem_g[k])
        pltpu.async_copy(haug_hbm.at[idx_v.at[k, 0, pl.ds(hc, hc)]],
                         buf_v.at[k, pl.ds(hc, hc)], sem_g2[k])

    def _drain_scatter(k):
        pltpu.make_async_copy(buf_v.at[k], agg_s.at[idx_v.at[k, 1]],
                              sem_s[k]).wait()

    for k in range(2):
        _unpack(k, k)
        _start_gather(k)

    def _step(k, b, refill):
        hc = C // 2
        pltpu.make_async_copy(haug_hbm.at[idx_v.at[k, 0, pl.ds(0, hc)]],
                              buf_v.at[k, pl.ds(0, hc)], sem_g[k]).wait()
        pltpu.make_async_copy(haug_hbm.at[idx_v.at[k, 0, pl.ds(hc, hc)]],
                              buf_v.at[k, pl.ds(hc, hc)], sem_g2[k]).wait()

        def _relu(i, _):
            for j in range(H // 16):
                sl = buf_v[k, i, pl.ds(j * 16, 16)]
                buf_v[k, i, pl.ds(j * 16, 16)] = jnp.maximum(sl, 0.0)
            return 0
        lax.fori_loop(0, C, _relu, 0)
        pltpu.async_copy(buf_v.at[k], agg_s.at[idx_v.at[k, 1]],
                         sem_s[k], add=True)

        if refill:
            k2 = (k + 2) % NBUF

            @pl.when(b + 2 < CHUNKS)
            def _():
                @pl.when(b >= NBUF - 2)
                def _():
                    _drain_scatter(k2)
                _unpack(k2, b + 2)
                _start_gather(k2)

    def _group(g, _):
        for k in range(NBUF):
            _step(k, g * NBUF + k, True)
        return 0
    lax.fori_loop(0, GROUPS, _group, 0)
    for t in range(TAIL):
        _step((GROUPS * NBUF + t) % NBUF, GROUPS * NBUF + t, False)
    for t in range(NBUF):
        _drain_scatter((CHUNKS - NBUF + t) % NBUF)

    plsc.subcore_barrier()
    r0 = sid * ROWS_PT
    pltpu.sync_copy(agg_s.at[pl.ds(r0, ROWS_PT)],
                    agg_hbm.at[cid, pl.ds(r0, ROWS_PT)])


@functools.cache
def _get_sc_edge():
    return pl.kernel(
        _sc_edge_body,
        out_type=jax.ShapeDtypeStruct((NC, Np, H), jnp.float32),
        mesh=plsc.VectorSubcoreMesh(core_axis_name="c",
                                    subcore_axis_name="s",
                                    num_cores=NC, num_subcores=NS),
        scratch_types=(
            [pltpu.VMEM((CHUNKS, C), jnp.int32),
             pltpu.VMEM((NBUF, 2, C), jnp.int32),
             pltpu.VMEM((NBUF, C, H), jnp.float32),
             pltpu.VMEM_SHARED((Np, H), jnp.float32)]
            + [pltpu.SemaphoreType.DMA] * (3 * NBUF)
        ),
    )


def _embed_body(xn_ref, pe_ref, emb_ref, pew_ref, peb_ref, ea_ref,
                h_ref, haug_ref):
    x = xn_ref[...]
    iota = lax.broadcasted_iota(jnp.int32, (NB, VOCAB), 1)
    onehot = (x == iota).astype(jnp.float32)
    h = jnp.dot(onehot, emb_ref[...], preferred_element_type=jnp.float32,
                precision=lax.Precision.HIGHEST)
    h += jnp.dot(pe_ref[...], pew_ref[...], preferred_element_type=jnp.float32,
                precision=lax.Precision.HIGHEST)
    h += peb_ref[...]
    h_ref[...] = h
    haug_ref[...] = h[None, :, :] + ea_ref[...][:, None, :]


def _mlp_body_aug(h_ref, agg_ref, w1_ref, b1_ref, w2_ref, b2_ref,
                  sc_ref, ea_ref, ho_ref, haug_ref):
    z = sc_ref[0, 0] * h_ref[...] + agg_ref[0] + agg_ref[1]
    a = jnp.maximum(
        jnp.dot(z, w1_ref[...], preferred_element_type=jnp.float32,
                precision=lax.Precision.HIGHEST)
        + b1_ref[...], 0.0)
    hn = jnp.dot(a, w2_ref[...], preferred_element_type=jnp.float32,
                precision=lax.Precision.HIGHEST) \
        + b2_ref[...]
    ho_ref[...] = hn
    haug_ref[...] = hn[None, :, :] + ea_ref[...][:, None, :]


def _mlp_body_last(h_ref, agg_ref, w1_ref, b1_ref, w2_ref, b2_ref,
                   sc_ref, ho_ref):
    z = sc_ref[0, 0] * h_ref[...] + agg_ref[0] + agg_ref[1]
    a = jnp.maximum(
        jnp.dot(z, w1_ref[...], preferred_element_type=jnp.float32,
                precision=lax.Precision.HIGHEST)
        + b1_ref[...], 0.0)
    ho_ref[...] = jnp.dot(a, w2_ref[...],
                          preferred_element_type=jnp.float32,
                precision=lax.Precision.HIGHEST) + b2_ref[...]


def _pool_head_body(bt_ref, h_ref, f1_ref, fb1_ref, f2_ref, fb2_ref,
                    f3_ref, fb3_ref, y_ref, acc, cnt):
    i = pl.program_id(0)

    @pl.when(i == 0)
    def _init():
        acc[...] = jnp.zeros((G, H), jnp.float32)
        cnt[...] = jnp.zeros((G, 1), jnp.float32)

    b = bt_ref[...]
    iota = lax.broadcasted_iota(jnp.int32, (NB, G), 1)
    m = (b == iota).astype(jnp.float32)
    dn = (((0,), (0,)), ((), ()))
    acc[...] += lax.dot_general(m, h_ref[...], dn,
                                preferred_element_type=jnp.float32,
                precision=lax.Precision.HIGHEST)
    cnt[...] += lax.dot_general(m, jnp.ones((NB, 1), jnp.float32), dn,
                                preferred_element_type=jnp.float32,
                precision=lax.Precision.HIGHEST)

    @pl.when(i == NBLK - 1)
    def _final():
        hp = acc[...] / jnp.maximum(cnt[...], 1.0)
        y = jnp.maximum(
            jnp.dot(hp, f1_ref[...], preferred_element_type=jnp.float32,
                precision=lax.Precision.HIGHEST)
            + fb1_ref[...], 0.0)
        y = jnp.maximum(
            jnp.dot(y, f2_ref[...], preferred_element_type=jnp.float32,
                precision=lax.Precision.HIGHEST)
            + fb2_ref[...], 0.0)
        y_ref[...] = jnp.dot(y, f3_ref[...],
                             preferred_element_type=jnp.float32,
                precision=lax.Precision.HIGHEST) + fb3_ref[...]


def _full(shape):
    return pl.BlockSpec(shape, lambda i: (0,) * len(shape))


_row_i = pl.BlockSpec((NB, 1), lambda i: (i, 0))
_row_h = pl.BlockSpec((NB, H), lambda i: (i, 0))
_aug_b = pl.BlockSpec((NET, NB, H), lambda i: (0, i, 0))
_agg_b = pl.BlockSpec((NC, NB, H), lambda i: (0, i, 0))
_smem1 = pl.BlockSpec(memory_space=pltpu.SMEM)

_embed_call = pl.pallas_call(
    _embed_body,
    grid=(NBLK,),
    in_specs=[_row_i, _row_h, _full((VOCAB, H)), _full((H, H)),
              _full((1, H)), _full((NET, H))],
    out_specs=[_row_h, _aug_b],
    out_shape=[jax.ShapeDtypeStruct((Np, H), jnp.float32),
               jax.ShapeDtypeStruct((NET, Np, H), jnp.float32)],
)

_mlp_aug_call = pl.pallas_call(
    _mlp_body_aug,
    grid=(NBLK,),
    in_specs=[_row_h, _agg_b, _full((H, H)), _full((1, H)), _full((H, H)),
              _full((1, H)), _smem1, _full((NET, H))],
    out_specs=[_row_h, _aug_b],
    out_shape=[jax.ShapeDtypeStruct((Np, H), jnp.float32),
               jax.ShapeDtypeStruct((NET, Np, H), jnp.float32)],
)

_mlp_last_call = pl.pallas_call(
    _mlp_body_last,
    grid=(NBLK,),
    in_specs=[_row_h, _agg_b, _full((H, H)), _full((1, H)), _full((H, H)),
              _full((1, H)), _smem1],
    out_specs=_row_h,
    out_shape=jax.ShapeDtypeStruct((Np, H), jnp.float32),
)

_pool_head_call = pl.pallas_call(
    _pool_head_body,
    grid=(NBLK,),
    in_specs=[_row_i, _row_h, _full((H, H)), _full((1, H)), _full((H, H)),
              _full((1, H)), _full((H, 1)), _full((1, 1))],
    out_specs=pl.BlockSpec((G, 1), lambda i: (0, 0)),
    out_shape=jax.ShapeDtypeStruct((G, 1), jnp.float32),
    scratch_shapes=[pltpu.VMEM((G, H), jnp.float32),
                    pltpu.VMEM((G, 1), jnp.float32)],
)


def kernel(X_n, edge_index, edge_attr, PE, snorm, batch, node_emb, pe_W,
           pe_b, edge_embs, W1, b1, W2, b2, eps, fW1, fb1, fW2, fb2, fW3,
           fb3):
    i32 = jnp.int32
    src = edge_index[0].astype(i32)
    dst = edge_index[1].astype(i32)
    attr = edge_attr.astype(i32)
    gidx = attr * Np + src
    pad_e = E_PAD - E
    gidx = jnp.concatenate([gidx, jnp.zeros((pad_e,), i32)])
    dst = jnp.concatenate([dst, jnp.full((pad_e,), N, i32)])
    ipack = ((dst << 16) | gidx).reshape(NCHT, C)

    xn = jnp.concatenate([X_n.astype(i32),
                          jnp.zeros((Np - N,), i32)]).reshape(Np, 1)
    pe = jnp.zeros((Np, H), jnp.float32).at[:N, :PED].set(PE)
    pew = jnp.zeros((H, H), jnp.float32).at[:PED].set(pe_W)
    bt = jnp.concatenate([batch.astype(i32),
                          jnp.full((Np - N,), G, i32)]).reshape(Np, 1)

    h, haug = _embed_call(xn, pe, node_emb, pew, pe_b.reshape(1, H),
                          edge_embs[0])
    for l in range(L):
        agg = _get_sc_edge()(ipack, haug.reshape(NET * Np, H))
        scale = (1.0 + eps[l]).reshape(1, 1)
        if l + 1 < L:
            h, haug = _mlp_aug_call(h, agg, W1[l], b1[l].reshape(1, H),
                                    W2[l], b2[l].reshape(1, H), scale,
                                    edge_embs[l + 1])
        else:
            h = _mlp_last_call(h, agg, W1[l], b1[l].reshape(1, H), W2[l],
                               b2[l].reshape(1, H), scale)

    y = _pool_head_call(bt, h, fW1, fb1.reshape(1, H), fW2,
                        fb2.reshape(1, H), fW3, fb3.reshape(1, 1))
    return y[:, 0]
